# baseline (speedup 1.0000x reference)
import functools

import jax
import jax.numpy as jnp
from jax.experimental import pallas as pl
from jax.experimental.pallas import tpu as pltpu

_BN = 1000


def _fused_linears_kernel(x_ref, wc_ref, bc_ref, wb_ref, bb_ref,
                          scores_ref, deltas_ref):
    x = x_ref[...]
    scores_ref[...] = (
        jnp.dot(x, wc_ref[...], preferred_element_type=jnp.float32)
        + bc_ref[...]
    )
    deltas_ref[...] = (
        jnp.dot(x, wb_ref[...], preferred_element_type=jnp.float32)
        + bb_ref[...]
    )


@jax.jit
def kernel(x, W_cls, b_cls, W_box, b_box):
    if x.ndim > 2:
        x = x.reshape((x.shape[0], -1))
    n, d = x.shape
    kc = W_cls.shape[1]
    kb = W_box.shape[1]
    grid = (n // _BN,)
    scores, deltas = pl.pallas_call(
        _fused_linears_kernel,
        grid=grid,
        in_specs=[
            pl.BlockSpec((_BN, d), lambda i: (i, 0)),
            pl.BlockSpec((d, kc), lambda i: (0, 0)),
            pl.BlockSpec((kc,), lambda i: (0,)),
            pl.BlockSpec((d, kb), lambda i: (0, 0)),
            pl.BlockSpec((kb,), lambda i: (0,)),
        ],
        out_specs=[
            pl.BlockSpec((_BN, kc), lambda i: (i, 0)),
            pl.BlockSpec((_BN, kb), lambda i: (i, 0)),
        ],
        out_shape=[
            jax.ShapeDtypeStruct((n, kc), jnp.float32),
            jax.ShapeDtypeStruct((n, kb), jnp.float32),
        ],
        compiler_params=pltpu.CompilerParams(
            dimension_semantics=("parallel",),
        ),
    )(x, W_cls, b_cls, W_box, b_box)
    return (scores, deltas)
